# baseline (speedup 1.0000x reference)
import jax
import jax.numpy as jnp
from jax import lax
from jax.experimental import pallas as pl
from jax.experimental.pallas import tpu as pltpu
from jax.experimental.pallas import tpu_sc as plsc

_VOCAB = 100000
_EMB = 128
_DMODEL = 1024
_EPS = 1e-5

_BATCH = 4
_SEQ = 2048
_ROWS = _BATCH * _SEQ
_NW = 32
_RPW = _ROWS // _NW
_CHUNK = 128
_NCH = _RPW // _CHUNK


def _gather_body(idx_hbm, table_hbm, out_hbm, idx_v, rows_v, gsem, wsem):
    c = lax.axis_index("c")
    s = lax.axis_index("s")
    wid = s * 2 + c
    pltpu.sync_copy(idx_hbm.at[pl.ds(wid * _NCH, _NCH)], idx_v)
    gathers = [
        pltpu.async_copy(
            table_hbm.at[idx_v.at[j]],
            rows_v.at[pl.ds(j * _CHUNK, _CHUNK)],
            gsem,
        )
        for j in range(_NCH)
    ]
    writes = []
    for j in range(_NCH):
        gathers[j].wait()
        writes.append(
            pltpu.async_copy(
                rows_v.at[pl.ds(j * _CHUNK, _CHUNK)],
                out_hbm.at[pl.ds(wid * _RPW + j * _CHUNK, _CHUNK)],
                wsem,
            )
        )
    for w in writes:
        w.wait()


def _make_gather():
    return pl.kernel(
        _gather_body,
        mesh=plsc.VectorSubcoreMesh(core_axis_name="c", subcore_axis_name="s"),
        out_type=jax.ShapeDtypeStruct((_ROWS, _EMB), jnp.float32),
        scratch_types=[
            pltpu.VMEM((_NCH, _CHUNK), jnp.int32),
            pltpu.VMEM((_RPW, _EMB), jnp.float32),
            pltpu.SemaphoreType.DMA,
            pltpu.SemaphoreType.DMA,
        ],
    )


_T = 2048
_NSTR = 4
_SROWS = _T // _NSTR


def _dense_body(te_ref, seg_ref, pe_ref, st_ref, g_ref, be_ref, w_ref, b_ref,
                o_hbm, sc_ref, sems):
    i = pl.program_id(0)
    prev = lax.max(i - 2, 0)

    def waits(bufc, step):
        for k in range(_NSTR):
            pltpu.make_async_copy(
                sc_ref.at[bufc].at[pl.ds(k * _SROWS, _SROWS)],
                o_hbm.at[step].at[pl.ds(k * _SROWS, _SROWS)],
                sems.at[bufc, k],
            ).wait()

    def fires(bufc, step):
        for k in range(_NSTR):
            pltpu.make_async_copy(
                sc_ref.at[bufc].at[pl.ds(k * _SROWS, _SROWS)],
                o_hbm.at[step].at[pl.ds(k * _SROWS, _SROWS)],
                sems.at[bufc, k],
            ).start()

    buf = lax.rem(i, 2)

    @pl.when(jnp.logical_and(i >= 2, buf == 0))
    def _():
        waits(0, prev)

    @pl.when(jnp.logical_and(i >= 2, buf == 1))
    def _():
        waits(1, prev)

    te = te_ref[0]
    segf = seg_ref[0].astype(jnp.float32)
    s0 = st_ref[0:1, :]
    s1 = st_ref[1:2, :]
    hs = te + pe_ref[...] + s0 + segf * (s1 - s0)
    mu = jnp.mean(hs, axis=1, keepdims=True)
    d = hs - mu
    var = jnp.mean(d * d, axis=1, keepdims=True)
    hsn = d * lax.rsqrt(var + _EPS) * g_ref[...] + be_ref[...]
    r = jnp.dot(hsn, w_ref[...], preferred_element_type=jnp.float32) + b_ref[...]

    @pl.when(buf == 0)
    def _():
        sc_ref[0] = r
        fires(0, i)

    @pl.when(buf == 1)
    def _():
        sc_ref[1] = r
        fires(1, i)

    @pl.when(i == _BATCH - 1)
    def _():
        waits((_BATCH - 2) % 2, _BATCH - 2)
        waits((_BATCH - 1) % 2, _BATCH - 1)


def _dense(te, seg, pos_table, seg_table, gamma, beta, w, b):
    grid = (_BATCH,)
    return pl.pallas_call(
        _dense_body,
        grid=grid,
        in_specs=[
            pl.BlockSpec((1, _T, _EMB), lambda i: (i, 0, 0)),
            pl.BlockSpec((1, _T, 1), lambda i: (i, 0, 0)),
            pl.BlockSpec((_T, _EMB), lambda i: (0, 0)),
            pl.BlockSpec((2, _EMB), lambda i: (0, 0)),
            pl.BlockSpec((1, _EMB), lambda i: (0, 0)),
            pl.BlockSpec((1, _EMB), lambda i: (0, 0)),
            pl.BlockSpec((_EMB, _DMODEL), lambda i: (0, 0)),
            pl.BlockSpec((1, _DMODEL), lambda i: (0, 0)),
        ],
        out_specs=pl.BlockSpec(memory_space=pl.ANY),
        out_shape=jax.ShapeDtypeStruct((_BATCH, _SEQ, _DMODEL), jnp.float32),
        scratch_shapes=[
            pltpu.VMEM((2, _T, _DMODEL), jnp.float32),
            pltpu.SemaphoreType.DMA((2, _NSTR)),
        ],
    )(te, seg, pos_table, seg_table, gamma, beta, w, b)


def kernel(tokens, segments, token_table, seg_table, pos_table, gamma, beta, W, b):
    idx = tokens.reshape(_ROWS // _CHUNK, _CHUNK)
    te = _make_gather()(idx, token_table)
    return _dense(
        te.reshape(_BATCH, _SEQ, _EMB),
        segments.reshape(_BATCH, _SEQ, 1),
        pos_table,
        seg_table,
        gamma.reshape(1, _EMB),
        beta.reshape(1, _EMB),
        W,
        b.reshape(1, _DMODEL),
    )


# speedup vs baseline: 1.0262x; 1.0262x over previous
import jax
import jax.numpy as jnp
from jax import lax
from jax.experimental import pallas as pl
from jax.experimental.pallas import tpu as pltpu
from jax.experimental.pallas import tpu_sc as plsc

_VOCAB = 100000
_EMB = 128
_DMODEL = 1024
_EPS = 1e-5

_BATCH = 4
_SEQ = 2048
_ROWS = _BATCH * _SEQ
_NW = 32
_RPW = _ROWS // _NW
_CHUNK = 128
_NCH = _RPW // _CHUNK


def _gather_body(idx_hbm, table_hbm, out_hbm, idx_v, rows_v, gsem, wsem):
    c = lax.axis_index("c")
    s = lax.axis_index("s")
    wid = s * 2 + c
    pltpu.sync_copy(idx_hbm.at[pl.ds(wid * _NCH, _NCH)], idx_v)
    gathers = [
        pltpu.async_copy(
            table_hbm.at[idx_v.at[j]],
            rows_v.at[pl.ds(j * _CHUNK, _CHUNK)],
            gsem,
        )
        for j in range(_NCH)
    ]
    writes = []
    for j in range(_NCH):
        gathers[j].wait()
        writes.append(
            pltpu.async_copy(
                rows_v.at[pl.ds(j * _CHUNK, _CHUNK)],
                out_hbm.at[pl.ds(wid * _RPW + j * _CHUNK, _CHUNK)],
                wsem,
            )
        )
    for w in writes:
        w.wait()


def _make_gather():
    return pl.kernel(
        _gather_body,
        mesh=plsc.VectorSubcoreMesh(core_axis_name="c", subcore_axis_name="s"),
        out_type=jax.ShapeDtypeStruct((_ROWS, _EMB), jnp.float32),
        scratch_types=[
            pltpu.VMEM((_NCH, _CHUNK), jnp.int32),
            pltpu.VMEM((_RPW, _EMB), jnp.float32),
            pltpu.SemaphoreType.DMA,
            pltpu.SemaphoreType.DMA,
        ],
    )


_T = 2048


def _dense_body(te_ref, seg_ref, pe_ref, st_ref, g_ref, be_ref, w_ref, b_ref, o_ref):
    te = te_ref[0]
    segf = seg_ref[0].astype(jnp.float32)
    s0 = st_ref[0:1, :]
    s1 = st_ref[1:2, :]
    hs = te + pe_ref[...] + s0 + segf * (s1 - s0)
    mu = jnp.mean(hs, axis=1, keepdims=True)
    d = hs - mu
    var = jnp.mean(d * d, axis=1, keepdims=True)
    hsn = d * lax.rsqrt(var + _EPS) * g_ref[...] + be_ref[...]
    o_ref[0] = (
        jnp.dot(hsn, w_ref[...], preferred_element_type=jnp.float32) + b_ref[...]
    )


def _dense(te, seg, pos_table, seg_table, gamma, beta, w, b):
    grid = (_BATCH,)
    return pl.pallas_call(
        _dense_body,
        grid=grid,
        in_specs=[
            pl.BlockSpec((1, _T, _EMB), lambda i: (i, 0, 0)),
            pl.BlockSpec((1, _T, 1), lambda i: (i, 0, 0)),
            pl.BlockSpec((_T, _EMB), lambda i: (0, 0)),
            pl.BlockSpec((2, _EMB), lambda i: (0, 0)),
            pl.BlockSpec((1, _EMB), lambda i: (0, 0)),
            pl.BlockSpec((1, _EMB), lambda i: (0, 0)),
            pl.BlockSpec((_EMB, _DMODEL), lambda i: (0, 0)),
            pl.BlockSpec((1, _DMODEL), lambda i: (0, 0)),
        ],
        out_specs=pl.BlockSpec((1, _T, _DMODEL), lambda i: (i, 0, 0)),
        out_shape=jax.ShapeDtypeStruct((_BATCH, _SEQ, _DMODEL), jnp.float32),
    )(te, seg, pos_table, seg_table, gamma, beta, w, b)


def kernel(tokens, segments, token_table, seg_table, pos_table, gamma, beta, W, b):
    idx = tokens.reshape(_ROWS // _CHUNK, _CHUNK)
    te = _make_gather()(idx, token_table)
    return _dense(
        te.reshape(_BATCH, _SEQ, _EMB),
        segments.reshape(_BATCH, _SEQ, 1),
        pos_table,
        seg_table,
        gamma.reshape(1, _EMB),
        beta.reshape(1, _EMB),
        W,
        b.reshape(1, _DMODEL),
    )


# speedup vs baseline: 1.0339x; 1.0075x over previous
import jax
import jax.numpy as jnp
from jax import lax
from jax.experimental import pallas as pl
from jax.experimental.pallas import tpu as pltpu
from jax.experimental.pallas import tpu_sc as plsc

_VOCAB = 100000
_EMB = 128
_DMODEL = 1024
_EPS = 1e-5

_BATCH = 4
_SEQ = 2048
_ROWS = _BATCH * _SEQ
_NW = 32
_RPW = _ROWS // _NW
_CHUNK = 128
_NCH = _RPW // _CHUNK


def _gather_body(idx_hbm, table_hbm, out_hbm, idx_v, rows_v, sem):
    c = lax.axis_index("c")
    s = lax.axis_index("s")
    wid = s * 2 + c
    pltpu.sync_copy(idx_hbm.at[pl.ds(wid * _NCH, _NCH)], idx_v)
    copies = [
        pltpu.async_copy(
            table_hbm.at[idx_v.at[j]],
            rows_v.at[pl.ds(j * _CHUNK, _CHUNK)],
            sem,
        )
        for j in range(_NCH)
    ]
    for cp in copies:
        cp.wait()
    pltpu.sync_copy(rows_v, out_hbm.at[pl.ds(wid * _RPW, _RPW)])


def _make_gather():
    return pl.kernel(
        _gather_body,
        mesh=plsc.VectorSubcoreMesh(core_axis_name="c", subcore_axis_name="s"),
        out_type=jax.ShapeDtypeStruct((_ROWS, _EMB), jnp.float32),
        scratch_types=[
            pltpu.VMEM((_NCH, _CHUNK), jnp.int32),
            pltpu.VMEM((_RPW, _EMB), jnp.float32),
            pltpu.SemaphoreType.DMA,
        ],
    )


_T = 2048


def _dense_body(te_ref, seg_ref, pe_ref, st_ref, g_ref, be_ref, w_ref, b_ref, o_ref):
    te = te_ref[0]
    segf = seg_ref[0].astype(jnp.float32)
    s0 = st_ref[0:1, :]
    s1 = st_ref[1:2, :]
    hs = te + pe_ref[...] + s0 + segf * (s1 - s0)
    mu = jnp.mean(hs, axis=1, keepdims=True)
    d = hs - mu
    var = jnp.mean(d * d, axis=1, keepdims=True)
    hsn = d * lax.rsqrt(var + _EPS) * g_ref[...] + be_ref[...]
    o_ref[0] = (
        jnp.dot(hsn, w_ref[...], preferred_element_type=jnp.float32) + b_ref[...]
    )


def _dense(te, seg, pos_table, seg_table, gamma, beta, w, b):
    grid = (_SEQ // _T, _BATCH)
    return pl.pallas_call(
        _dense_body,
        grid=grid,
        in_specs=[
            pl.BlockSpec((1, _T, _EMB), lambda j, i: (i, j, 0)),
            pl.BlockSpec((1, _T, 1), lambda j, i: (i, j, 0)),
            pl.BlockSpec((_T, _EMB), lambda j, i: (j, 0)),
            pl.BlockSpec((2, _EMB), lambda j, i: (0, 0)),
            pl.BlockSpec((1, _EMB), lambda j, i: (0, 0)),
            pl.BlockSpec((1, _EMB), lambda j, i: (0, 0)),
            pl.BlockSpec((_EMB, _DMODEL), lambda j, i: (0, 0)),
            pl.BlockSpec((1, _DMODEL), lambda j, i: (0, 0)),
        ],
        out_specs=pl.BlockSpec((1, _T, _DMODEL), lambda j, i: (i, j, 0)),
        out_shape=jax.ShapeDtypeStruct((_BATCH, _SEQ, _DMODEL), jnp.float32),
    )(te, seg, pos_table, seg_table, gamma, beta, w, b)


def kernel(tokens, segments, token_table, seg_table, pos_table, gamma, beta, W, b):
    idx = tokens.reshape(_ROWS // _CHUNK, _CHUNK)
    te = _make_gather()(idx, token_table)
    return _dense(
        te.reshape(_BATCH, _SEQ, _EMB),
        segments.reshape(_BATCH, _SEQ, 1),
        pos_table,
        seg_table,
        gamma.reshape(1, _EMB),
        beta.reshape(1, _EMB),
        W,
        b.reshape(1, _DMODEL),
    )
